# baseline (speedup 1.0000x reference)
import functools

import jax
import jax.numpy as jnp
from jax import lax
from jax.experimental import pallas as pl
from jax.experimental.pallas import tpu as pltpu
from jax.experimental.pallas import tpu_sc as plsc

NEAR = 0.01
FAR = 5000.0
_E = FAR / (FAR - NEAR)
_F = -(FAR * NEAR) / (FAR - NEAR)
_FB = float(jnp.bfloat16(jnp.float32(_F)))

_CH = 128


def _sc_gather(table8, idxq, B, b_per_w, nch):
    mesh = plsc.VectorSubcoreMesh(core_axis_name="c", subcore_axis_name="s")
    NC = 2

    scratch = [pltpu.VMEM((_CH,), jnp.int32) for _ in range(nch)]
    scratch += [pltpu.VMEM((_CH, 8), jnp.float32) for _ in range(nch)]
    scratch += [pltpu.SemaphoreType.DMA]

    @functools.partial(
        pl.kernel,
        mesh=mesh,
        out_type=jax.ShapeDtypeStruct((B, 8), jnp.float32),
        scratch_types=scratch,
        compiler_params=pltpu.CompilerParams(use_tc_tiling_on_sc=False,
                                             needs_layout_passes=False),
    )
    def k(table_hbm, idxq_hbm, out_hbm, *sc):
        idxv = sc[:nch]
        rows_v = sc[nch:2 * nch]
        sem = sc[2 * nch]
        wid = lax.axis_index("s") * NC + lax.axis_index("c")
        base = wid * b_per_w
        for j in range(nch):
            pltpu.sync_copy(idxq_hbm.at[wid * nch + j], idxv[j])
        copies = [
            pltpu.async_copy(table_hbm.at[idxv[j]], rows_v[j], sem)
            for j in range(nch)
        ]
        for cp in copies:
            cp.wait()
        for j in range(nch):
            pltpu.sync_copy(rows_v[j], out_hbm.at[pl.ds(base + j * _CH, _CH)])

    return k(table8, idxq)


def _tc_math(comp, params, S, L):
    GRID = 8
    bs = S // GRID

    def bf(x):
        return x.astype(jnp.bfloat16).astype(jnp.float32)

    def body(comp_ref, p_ref, view_ref, vp_ref, fp_ref):
        rx = comp_ref[0]
        ry = comp_ref[1]
        rz = comp_ref[2]
        tx = comp_ref[3]
        ty = comp_ref[4]
        tz = comp_ref[5]

        theta = jnp.sqrt(rx * rx + ry * ry + rz * rz)
        den = theta + 1e-8
        kx = rx / den
        ky = ry / den
        kz = rz / den
        s = jnp.sin(theta)
        omc = 1.0 - jnp.cos(theta)

        kxB = bf(kx)
        kyB = bf(ky)
        kzB = bf(kz)
        k2_00 = -(kzB * kzB) - kyB * kyB
        k2_11 = -(kzB * kzB) - kxB * kxB
        k2_22 = -(kyB * kyB) - kxB * kxB
        k2_xy = kxB * kyB
        k2_xz = kxB * kzB
        k2_yz = kyB * kzB

        r00 = 1.0 + omc * k2_00
        r01 = (s * -kz) + omc * k2_xy
        r02 = (s * ky) + omc * k2_xz
        r10 = (s * kz) + omc * k2_xy
        r11 = 1.0 + omc * k2_11
        r12 = (s * -kx) + omc * k2_yz
        r20 = (s * -ky) + omc * k2_xz
        r21 = (s * kx) + omc * k2_yz
        r22 = 1.0 + omc * k2_22

        zero = jnp.zeros_like(rx)
        one = jnp.ones_like(rx)

        v = (r00, r01, r02, tx,
             r10, r11, r12, ty,
             r20, r21, r22, tz,
             zero, zero, zero, one)
        for i in range(16):
            view_ref[i] = v[i]

        aB = p_ref[0]
        bB = p_ref[1]
        cB = p_ref[2]
        dB = p_ref[3]

        r00B = bf(r00)
        r01B = bf(r01)
        r02B = bf(r02)
        r10B = bf(r10)
        r11B = bf(r11)
        r12B = bf(r12)
        r20B = bf(r20)
        r21B = bf(r21)
        r22B = bf(r22)
        txB = bf(tx)
        tyB = bf(ty)
        tzB = bf(tz)

        p00 = aB * r00B + cB * r20B
        p01 = aB * r01B + cB * r21B
        p02 = aB * r02B + cB * r22B
        p03 = aB * txB + cB * tzB
        p10 = bB * r10B + dB * r20B
        p11 = bB * r11B + dB * r21B
        p12 = bB * r12B + dB * r22B
        p13 = bB * tyB + dB * tzB
        p20 = r20B
        p21 = r21B
        p22 = r22B
        p23 = tzB + _FB
        p30 = r20B
        p31 = r21B
        p32 = r22B
        p33 = tzB

        vp = (p00, p01, p02, p03,
              p10, p11, p12, p13,
              p20, p21, p22, p23,
              p30, p31, p32, p33)
        for i in range(16):
            vp_ref[i] = vp[i]

        i = 0
        for (qx, qy, qz, qw) in (
            (p30 + p00, p31 + p01, p32 + p02, p33 + p03),
            (p30 - p00, p31 - p01, p32 - p02, p33 - p03),
            (p30 + p10, p31 + p11, p32 + p12, p33 + p13),
            (p30 - p10, p31 - p11, p32 - p12, p33 - p13),
            (p30 + p20, p31 + p21, p32 + p22, p33 + p23),
            (p30 - p20, p31 - p21, p32 - p22, p33 - p23),
        ):
            n = jnp.sqrt(qx * qx + qy * qy + qz * qz) + 1e-8
            fp_ref[i] = qx / n
            fp_ref[i + 1] = qy / n
            fp_ref[i + 2] = qz / n
            fp_ref[i + 3] = qw / n
            i += 4

    f32 = jnp.float32
    return pl.pallas_call(
        body,
        grid=(GRID,),
        in_specs=[
            pl.BlockSpec((6, bs, L), lambda i: (0, i, 0)),
            pl.BlockSpec(memory_space=pltpu.SMEM),
        ],
        out_specs=[
            pl.BlockSpec((16, bs, L), lambda i: (0, i, 0)),
            pl.BlockSpec((16, bs, L), lambda i: (0, i, 0)),
            pl.BlockSpec((24, bs, L), lambda i: (0, i, 0)),
        ],
        out_shape=[
            jax.ShapeDtypeStruct((16, S, L), f32),
            jax.ShapeDtypeStruct((16, S, L), f32),
            jax.ShapeDtypeStruct((24, S, L), f32),
        ],
    )(comp, params)


def kernel(idx, img_h, img_w, extr_weight, intrinsics):
    B = idx.shape[0]
    NW = 32
    b_per_w = B // NW
    nch = b_per_w // _CH

    table8 = jnp.pad(extr_weight, ((0, 0), (0, 2)))
    idxq = idx.astype(jnp.int32).reshape(NW * nch, _CH)
    g8 = _sc_gather(table8, idxq, B, b_per_w, nch)
    gathered = g8[:, :6].T.reshape(6 * B // _CH, _CH)

    S = 128
    L = B // S
    comp = gathered.reshape(6, S, L)

    fx, fy, cx, cy = (intrinsics[0, 0], intrinsics[0, 1],
                      intrinsics[0, 2], intrinsics[0, 3])
    W = jnp.asarray(img_w).astype(jnp.float32)
    H = jnp.asarray(img_h).astype(jnp.float32)
    a = 2.0 * fx / W
    b = 2.0 * fy / H
    c = 2.0 * cx / W - 1.0
    dd = 2.0 * cy / H - 1.0
    params = jnp.stack([a, b, c, dd]).astype(jnp.bfloat16).astype(jnp.float32)

    view16, vp16, fp24 = _tc_math(comp, params, S, L)

    view = view16.reshape(16, B).T.reshape(B, 4, 4)
    viewproj = vp16.reshape(16, B).T.reshape(B, 4, 4)
    frustumplane = fp24.reshape(24, B).T.reshape(B, 6, 4)

    proj = jnp.zeros((4, 4), dtype=jnp.float32)
    proj = proj.at[0, 0].set(2.0 * fx / W)
    proj = proj.at[1, 1].set(2.0 * fy / H)
    proj = proj.at[0, 2].set(2.0 * cx / W - 1.0)
    proj = proj.at[1, 2].set(2.0 * cy / H - 1.0)
    proj = proj.at[2, 2].set(_E)
    proj = proj.at[2, 3].set(_F)
    proj = proj.at[3, 2].set(1.0)
    proj_b = jnp.broadcast_to(proj, (B, 4, 4))

    return (view, proj_b, viewproj, frustumplane)


# speedup vs baseline: 1.4453x; 1.4453x over previous
import functools

import jax
import jax.numpy as jnp
from jax import lax
from jax.experimental import pallas as pl
from jax.experimental.pallas import tpu as pltpu
from jax.experimental.pallas import tpu_sc as plsc

NEAR = 0.01
FAR = 5000.0
_E = FAR / (FAR - NEAR)
_F = -(FAR * NEAR) / (FAR - NEAR)
_FB = float(jnp.bfloat16(jnp.float32(_F)))

_CH = 128


def _sc_gather(table2, idxr0, idxr1, idxc0, B, b_per_w, nch):
    mesh = plsc.VectorSubcoreMesh(core_axis_name="c", subcore_axis_name="s")
    NC = 2
    rows_per_plane = B // _CH

    scratch = [pltpu.VMEM((_CH,), jnp.int32) for _ in range(3)]
    scratch += [pltpu.VMEM((_CH, 128), jnp.float32) for _ in range(2)]
    scratch += [pltpu.VMEM((6, _CH), jnp.float32),
                pltpu.SemaphoreType.DMA]

    @functools.partial(
        pl.kernel,
        mesh=mesh,
        out_type=jax.ShapeDtypeStruct((6 * B // _CH, _CH), jnp.float32),
        scratch_types=scratch,
        compiler_params=pltpu.CompilerParams(use_tc_tiling_on_sc=False,
                                             needs_layout_passes=False),
    )
    def k(table_hbm, idxr0_hbm, idxr1_hbm, idxc0_hbm, out_hbm, *sc):
        r0v, r1v, c0v = sc[0], sc[1], sc[2]
        ga, gb = sc[3], sc[4]
        plane_v = sc[5]
        sem = sc[6]
        wid = lax.axis_index("s") * NC + lax.axis_index("c")
        lanes = jnp.arange(16, dtype=jnp.int32)
        for j in range(nch):
            row = wid * nch + j
            pltpu.sync_copy(idxr0_hbm.at[row], r0v)
            pltpu.sync_copy(idxr1_hbm.at[row], r1v)
            pltpu.sync_copy(idxc0_hbm.at[row], c0v)
            ca = pltpu.async_copy(table_hbm.at[r0v], ga, sem)
            cb = pltpu.async_copy(table_hbm.at[r1v], gb, sem)
            ca.wait()
            cb.wait()
            for v in range(_CH // 16):
                rr = lanes + (16 * v)
                c0 = c0v[pl.ds(16 * v, 16)]
                for c in range(6):
                    cc = c0 + c
                    cm = cc & 127
                    va = plsc.load_gather(ga, [rr, cm])
                    vb = plsc.load_gather(gb, [rr, cm])
                    plane_v[c, pl.ds(16 * v, 16)] = jnp.where(cc < 128, va, vb)
            for c in range(6):
                pltpu.sync_copy(
                    plane_v.at[pl.ds(c, 1)],
                    out_hbm.at[pl.ds(c * rows_per_plane + row, 1)])

    return k(table2, idxr0, idxr1, idxc0)


def _tc_math(comp, params, S, L):
    GRID = 8
    bs = S // GRID

    def bf(x):
        return x.astype(jnp.bfloat16).astype(jnp.float32)

    def body(comp_ref, p_ref, view_ref, vp_ref, fp_ref):
        rx = comp_ref[0]
        ry = comp_ref[1]
        rz = comp_ref[2]
        tx = comp_ref[3]
        ty = comp_ref[4]
        tz = comp_ref[5]

        theta = jnp.sqrt(rx * rx + ry * ry + rz * rz)
        den = theta + 1e-8
        kx = rx / den
        ky = ry / den
        kz = rz / den
        s = jnp.sin(theta)
        omc = 1.0 - jnp.cos(theta)

        kxB = bf(kx)
        kyB = bf(ky)
        kzB = bf(kz)
        k2_00 = -(kzB * kzB) - kyB * kyB
        k2_11 = -(kzB * kzB) - kxB * kxB
        k2_22 = -(kyB * kyB) - kxB * kxB
        k2_xy = kxB * kyB
        k2_xz = kxB * kzB
        k2_yz = kyB * kzB

        r00 = 1.0 + omc * k2_00
        r01 = (s * -kz) + omc * k2_xy
        r02 = (s * ky) + omc * k2_xz
        r10 = (s * kz) + omc * k2_xy
        r11 = 1.0 + omc * k2_11
        r12 = (s * -kx) + omc * k2_yz
        r20 = (s * -ky) + omc * k2_xz
        r21 = (s * kx) + omc * k2_yz
        r22 = 1.0 + omc * k2_22

        zero = jnp.zeros_like(rx)
        one = jnp.ones_like(rx)

        v = (r00, r01, r02, tx,
             r10, r11, r12, ty,
             r20, r21, r22, tz,
             zero, zero, zero, one)
        for i in range(16):
            view_ref[i] = v[i]

        aB = p_ref[0]
        bB = p_ref[1]
        cB = p_ref[2]
        dB = p_ref[3]

        r00B = bf(r00)
        r01B = bf(r01)
        r02B = bf(r02)
        r10B = bf(r10)
        r11B = bf(r11)
        r12B = bf(r12)
        r20B = bf(r20)
        r21B = bf(r21)
        r22B = bf(r22)
        txB = bf(tx)
        tyB = bf(ty)
        tzB = bf(tz)

        p00 = aB * r00B + cB * r20B
        p01 = aB * r01B + cB * r21B
        p02 = aB * r02B + cB * r22B
        p03 = aB * txB + cB * tzB
        p10 = bB * r10B + dB * r20B
        p11 = bB * r11B + dB * r21B
        p12 = bB * r12B + dB * r22B
        p13 = bB * tyB + dB * tzB
        p20 = r20B
        p21 = r21B
        p22 = r22B
        p23 = tzB + _FB
        p30 = r20B
        p31 = r21B
        p32 = r22B
        p33 = tzB

        vp = (p00, p01, p02, p03,
              p10, p11, p12, p13,
              p20, p21, p22, p23,
              p30, p31, p32, p33)
        for i in range(16):
            vp_ref[i] = vp[i]

        i = 0
        for (qx, qy, qz, qw) in (
            (p30 + p00, p31 + p01, p32 + p02, p33 + p03),
            (p30 - p00, p31 - p01, p32 - p02, p33 - p03),
            (p30 + p10, p31 + p11, p32 + p12, p33 + p13),
            (p30 - p10, p31 - p11, p32 - p12, p33 - p13),
            (p30 + p20, p31 + p21, p32 + p22, p33 + p23),
            (p30 - p20, p31 - p21, p32 - p22, p33 - p23),
        ):
            n = jnp.sqrt(qx * qx + qy * qy + qz * qz) + 1e-8
            fp_ref[i] = qx / n
            fp_ref[i + 1] = qy / n
            fp_ref[i + 2] = qz / n
            fp_ref[i + 3] = qw / n
            i += 4

    f32 = jnp.float32
    return pl.pallas_call(
        body,
        grid=(GRID,),
        in_specs=[
            pl.BlockSpec((6, bs, L), lambda i: (0, i, 0)),
            pl.BlockSpec(memory_space=pltpu.SMEM),
        ],
        out_specs=[
            pl.BlockSpec((16, bs, L), lambda i: (0, i, 0)),
            pl.BlockSpec((16, bs, L), lambda i: (0, i, 0)),
            pl.BlockSpec((24, bs, L), lambda i: (0, i, 0)),
        ],
        out_shape=[
            jax.ShapeDtypeStruct((16, S, L), f32),
            jax.ShapeDtypeStruct((16, S, L), f32),
            jax.ShapeDtypeStruct((24, S, L), f32),
        ],
    )(comp, params)


def kernel(idx, img_h, img_w, extr_weight, intrinsics):
    B = idx.shape[0]
    NW = 32
    b_per_w = B // NW
    nch = b_per_w // _CH

    V = extr_weight.shape[0]
    table2 = extr_weight.reshape(V * 6 // 128, 128)
    idx32 = idx.astype(jnp.int32).reshape(NW * nch, _CH)
    e0 = 6 * idx32
    idxr0 = e0 // 128
    idxr1 = jnp.minimum(idxr0 + 1, V * 6 // 128 - 1)
    idxc0 = e0 % 128
    gathered = _sc_gather(table2, idxr0, idxr1, idxc0, B, b_per_w, nch)

    S = 128
    L = B // S
    comp = gathered.reshape(6, S, L)

    fx, fy, cx, cy = (intrinsics[0, 0], intrinsics[0, 1],
                      intrinsics[0, 2], intrinsics[0, 3])
    W = jnp.asarray(img_w).astype(jnp.float32)
    H = jnp.asarray(img_h).astype(jnp.float32)
    a = 2.0 * fx / W
    b = 2.0 * fy / H
    c = 2.0 * cx / W - 1.0
    dd = 2.0 * cy / H - 1.0
    params = jnp.stack([a, b, c, dd]).astype(jnp.bfloat16).astype(jnp.float32)

    view16, vp16, fp24 = _tc_math(comp, params, S, L)

    view = view16.reshape(16, B).T.reshape(B, 4, 4)
    viewproj = vp16.reshape(16, B).T.reshape(B, 4, 4)
    frustumplane = fp24.reshape(24, B).T.reshape(B, 6, 4)

    proj = jnp.zeros((4, 4), dtype=jnp.float32)
    proj = proj.at[0, 0].set(2.0 * fx / W)
    proj = proj.at[1, 1].set(2.0 * fy / H)
    proj = proj.at[0, 2].set(2.0 * cx / W - 1.0)
    proj = proj.at[1, 2].set(2.0 * cy / H - 1.0)
    proj = proj.at[2, 2].set(_E)
    proj = proj.at[2, 3].set(_F)
    proj = proj.at[3, 2].set(1.0)
    proj_b = jnp.broadcast_to(proj, (B, 4, 4))

    return (view, proj_b, viewproj, frustumplane)
